# Initial kernel scaffold; baseline (speedup 1.0000x reference)
import jax
import jax.numpy as jnp
from jax.experimental import pallas as pl


def kernel(x, edge_index, W, a):
    raise NotImplementedError("write your pallas kernel here")


# speedup vs baseline: 55.6493x; 55.6493x over previous
import functools

import jax
import jax.numpy as jnp
from jax import lax
from jax.experimental import pallas as pl
from jax.experimental.pallas import tpu as pltpu
from jax.experimental.pallas import tpu_sc as plsc

N = 10000
E = 320000
IN_DIM = 128
OUT_DIM = 32
NUM_HEADS = 4
D = NUM_HEADS * OUT_DIM

NC = 2
NS = 16
NW = NC * NS
CH = 128
NCHUNK = E // CH
RSTEP = 640
ZROWS = 80

_f32 = jnp.float32


def _tc_pre_body(x_ref, wc_ref, asrc_ref, adst_ref, zp_ref, es_ref, ed_ref,
                 m_ref):
    z = jnp.dot(x_ref[...], wc_ref[...], preferred_element_type=_f32)
    zp_ref[...] = z
    es = jnp.dot(z, asrc_ref[...], preferred_element_type=_f32)
    ed = jnp.dot(z, adst_ref[...], preferred_element_type=_f32)
    es_ref[...] = es
    ed_ref[...] = ed
    m = jnp.max(es, axis=0) + jnp.max(ed, axis=0)
    m = jnp.where(m > 0, m, 0.01 * m)
    m_ref[...] = jnp.broadcast_to(m[None, :], (8, 16))


def _tc_pre(x, wc, asrc_t, adst_t):
    return pl.pallas_call(
        _tc_pre_body,
        out_shape=(
            jax.ShapeDtypeStruct((N, D), _f32),
            jax.ShapeDtypeStruct((N, 16), _f32),
            jax.ShapeDtypeStruct((N, 16), _f32),
            jax.ShapeDtypeStruct((8, 16), _f32),
        ),
    )(x, wc, asrc_t, adst_t)


def _sc_body(zp_hbm, es_hbm, ed_hbm, m_hbm, src_hbm, dst_hbm,
             h_out, s_out,
             h_sh, s_sh,
             srcb, dstb, esb, edb, exb, zb, zero_h, zero_s, mb,
             sem_es, sem_ed, sem_z):
    cid = lax.axis_index("c")
    sid = lax.axis_index("s")
    wid = sid * NC + cid
    zeros16 = jnp.zeros((16,), _f32)

    def zzh(i, _):
        for g in range(8):
            zero_h[i, pl.ds(16 * g, 16)] = zeros16
        return 0

    def zzs(i, _):
        zero_s[i] = zeros16
        return 0

    lax.fori_loop(0, ZROWS, zzh, 0)
    lax.fori_loop(0, ZROWS, zzs, 0)
    row0 = sid * RSTEP
    nz = jnp.where(sid == NS - 1, 5, 8)

    def zcopy(r, _):
        pltpu.sync_copy(zero_h, h_sh.at[pl.ds(row0 + r * ZROWS, ZROWS), :])
        pltpu.sync_copy(zero_s, s_sh.at[pl.ds(row0 + r * ZROWS, ZROWS), :])
        return 0

    lax.fori_loop(0, nz, zcopy, 0)
    pltpu.sync_copy(m_hbm, mb)
    plsc.subcore_barrier()

    mvec = mb[0]
    nch = 78 + jnp.where(wid < NCHUNK - NW * (NCHUNK // NW), 1, 0)

    def chunk_body(i, _):
        eb = (i * NW + wid) * CH
        pltpu.sync_copy(src_hbm.at[pl.ds(eb, CH)], srcb)
        pltpu.sync_copy(dst_hbm.at[pl.ds(eb, CH)], dstb)
        cp_z = pltpu.make_async_copy(zp_hbm.at[srcb], zb, sem_z)
        cp_z.start()
        cp_es = pltpu.make_async_copy(es_hbm.at[srcb], esb, sem_es)
        cp_ed = pltpu.make_async_copy(ed_hbm.at[dstb], edb, sem_ed)
        cp_es.start()
        cp_ed.start()
        cp_es.wait()
        cp_ed.wait()

        def edge_body(e, _):
            v = esb[e] + edb[e]
            v = jnp.where(v > 0, v, 0.01 * v)
            exb[e] = jnp.exp(v - mvec)
            return 0

        lax.fori_loop(0, CH, edge_body, 0)
        cp_z.wait()

        def scale_body(e, _):
            ex = exb[e]
            for g in range(8):
                zb[e, pl.ds(16 * g, 16)] = zb[e, pl.ds(16 * g, 16)] * ex
            return 0

        lax.fori_loop(0, CH, scale_body, 0)
        pltpu.sync_copy(exb, s_sh.at[dstb], add=True)
        pltpu.sync_copy(zb, h_sh.at[dstb], add=True)
        return 0

    lax.fori_loop(0, nch, chunk_body, 0)
    plsc.subcore_barrier()

    @pl.when(sid < NS - 1)
    def _():
        pltpu.sync_copy(h_sh.at[pl.ds(row0, RSTEP), :],
                        h_out.at[cid, pl.ds(row0, RSTEP), :])
        pltpu.sync_copy(s_sh.at[pl.ds(row0, RSTEP), :],
                        s_out.at[cid, pl.ds(row0, RSTEP), :])

    @pl.when(sid == NS - 1)
    def _():
        pltpu.sync_copy(h_sh.at[pl.ds(row0, 400), :],
                        h_out.at[cid, pl.ds(row0, 400), :])
        pltpu.sync_copy(s_sh.at[pl.ds(row0, 400), :],
                        s_out.at[cid, pl.ds(row0, 400), :])


def _sc_edge_pass(zp, es, ed, m, src, dst):
    mesh = plsc.VectorSubcoreMesh(core_axis_name="c", subcore_axis_name="s")
    f = pl.kernel(
        _sc_body,
        out_type=(
            jax.ShapeDtypeStruct((NC, N, D), _f32),
            jax.ShapeDtypeStruct((NC, N, 16), _f32),
        ),
        mesh=mesh,
        scratch_types=[
            pltpu.VMEM_SHARED((N, D), _f32),
            pltpu.VMEM_SHARED((N, 16), _f32),
            pltpu.VMEM((CH,), jnp.int32),
            pltpu.VMEM((CH,), jnp.int32),
            pltpu.VMEM((CH, 16), _f32),
            pltpu.VMEM((CH, 16), _f32),
            pltpu.VMEM((CH, 16), _f32),
            pltpu.VMEM((CH, D), _f32),
            pltpu.VMEM((ZROWS, D), _f32),
            pltpu.VMEM((ZROWS, 16), _f32),
            pltpu.VMEM((8, 16), _f32),
            pltpu.SemaphoreType.DMA,
            pltpu.SemaphoreType.DMA,
            pltpu.SemaphoreType.DMA,
        ],
        compiler_params=pltpu.CompilerParams(use_tc_tiling_on_sc=False),
    )
    return f(zp, es, ed, m, src, dst)


def _tc_post_body(hp_ref, sp_ref, t2_ref, p_ref, out_ref):
    ht = hp_ref[0] + hp_ref[1]
    st = sp_ref[0] + sp_ref[1]
    r2 = jnp.dot(1.0 / (st + 1e-16), t2_ref[...],
                 preferred_element_type=_f32,
                 precision=jax.lax.Precision.HIGHEST)
    scaled = (ht * r2).reshape(N, D)
    out_ref[...] = jnp.dot(scaled, p_ref[...],
                           preferred_element_type=_f32,
                           precision=jax.lax.Precision.HIGHEST)


def _tc_post(hp, sp, t2, perm):
    return pl.pallas_call(
        _tc_post_body,
        out_shape=jax.ShapeDtypeStruct((N, D), _f32),
    )(hp, sp, t2, perm)


@jax.jit
def kernel(x, edge_index, W, a):
    src = edge_index[0]
    dst = edge_index[1]
    wc = jnp.transpose(W, (1, 2, 0)).reshape(IN_DIM, D)
    a_src = a[:, :OUT_DIM, 0]
    a_dst = a[:, OUT_DIM:, 0]
    j = jnp.arange(D)
    heads = j % NUM_HEADS
    dims = j // NUM_HEADS
    hot = (heads[:, None] == jnp.arange(NUM_HEADS)[None, :]).astype(_f32)
    asrc_p = a_src.T[dims] * hot
    adst_p = a_dst.T[dims] * hot
    asrc_t = jnp.tile(asrc_p, (1, 4))
    adst_t = jnp.tile(adst_p, (1, 4))
    perm = jax.nn.one_hot(heads * OUT_DIM + dims, D, dtype=_f32)
    mcol = jnp.arange(8 * D)
    t2 = jax.nn.one_hot(16 * (mcol // D) + (mcol % D) % NUM_HEADS,
                        D, dtype=_f32).T

    zp, es, ed, m = _tc_pre(x, wc, asrc_t, adst_t)
    hp, sp = _sc_edge_pass(zp, es, ed, m, src, dst)
    hp8 = hp.reshape(NC, N // 8, 8 * D)
    sp8 = sp.reshape(NC, N // 8, 8 * 16)
    return _tc_post(hp8, sp8, t2, perm)


# speedup vs baseline: 75.4826x; 1.3564x over previous
import functools

import jax
import jax.numpy as jnp
from jax import lax
from jax.experimental import pallas as pl
from jax.experimental.pallas import tpu as pltpu
from jax.experimental.pallas import tpu_sc as plsc

N = 10000
E = 320000
IN_DIM = 128
OUT_DIM = 32
NUM_HEADS = 4
D = NUM_HEADS * OUT_DIM

NC = 2
NS = 16
NW = NC * NS
CH = 80
NCHUNK = E // CH
RSTEP = 640
ZROWS = CH

_f32 = jnp.float32


def _tc_pre_body(x_ref, wc_ref, asrc_ref, adst_ref, zp_ref, es_ref, ed_ref,
                 m_ref):
    z = jnp.dot(x_ref[...], wc_ref[...], preferred_element_type=_f32)
    zp_ref[...] = z
    es = jnp.dot(z, asrc_ref[...], preferred_element_type=_f32)
    ed = jnp.dot(z, adst_ref[...], preferred_element_type=_f32)
    es_ref[...] = es
    ed_ref[...] = ed
    m = jnp.max(es, axis=0) + jnp.max(ed, axis=0)
    m = jnp.where(m > 0, m, 0.01 * m)
    m_ref[...] = jnp.broadcast_to(m[None, :], (8, 16))


def _tc_pre(x, wc, asrc_t, adst_t):
    return pl.pallas_call(
        _tc_pre_body,
        out_shape=(
            jax.ShapeDtypeStruct((N, D), _f32),
            jax.ShapeDtypeStruct((N, 16), _f32),
            jax.ShapeDtypeStruct((N, 16), _f32),
            jax.ShapeDtypeStruct((8, 16), _f32),
        ),
    )(x, wc, asrc_t, adst_t)


NCHW = NCHUNK // NW


def _sc_body(zp_hbm, es_hbm, ed_hbm, m_hbm, src_hbm, dst_hbm,
             h_out, s_out,
             h_sh, s_sh,
             srcb, dstb, esb, edb, exb, zb, mb,
             sem_es, sem_ed, sem_z, sem_s, sem_h):
    cid = lax.axis_index("c")
    sid = lax.axis_index("s")
    wid = sid * NC + cid
    zeros16 = jnp.zeros((16,), _f32)

    def zzh(i, _):
        for g in range(8):
            zb[0, i, pl.ds(16 * g, 16)] = zeros16
        esb[0, i] = zeros16
        return 0

    lax.fori_loop(0, ZROWS, zzh, 0)
    row0 = sid * RSTEP
    nz = jnp.where(sid == NS - 1, 5, 8)

    def zcopy(r, _):
        pltpu.sync_copy(zb.at[0], h_sh.at[pl.ds(row0 + r * ZROWS, ZROWS), :])
        pltpu.sync_copy(esb.at[0], s_sh.at[pl.ds(row0 + r * ZROWS, ZROWS), :])
        return 0

    lax.fori_loop(0, nz, zcopy, 0)
    pltpu.sync_copy(m_hbm, mb)
    plsc.subcore_barrier()

    mvec = mb[0]

    def _start_gathers(b, i):
        eb = (i * NW + wid) * CH
        pltpu.sync_copy(src_hbm.at[pl.ds(eb, CH)], srcb.at[b])
        pltpu.sync_copy(dst_hbm.at[pl.ds(eb, CH)], dstb.at[b])
        pltpu.make_async_copy(zp_hbm.at[srcb.at[b]], zb.at[b],
                              sem_z.at[b]).start()
        pltpu.make_async_copy(es_hbm.at[srcb.at[b]], esb.at[b],
                              sem_es.at[b]).start()
        pltpu.make_async_copy(ed_hbm.at[dstb.at[b]], edb.at[b],
                              sem_ed.at[b]).start()

    def _wait_gathers(b):
        pltpu.make_async_copy(zp_hbm.at[srcb.at[b]], zb.at[b],
                              sem_z.at[b]).wait()
        pltpu.make_async_copy(es_hbm.at[srcb.at[b]], esb.at[b],
                              sem_es.at[b]).wait()
        pltpu.make_async_copy(ed_hbm.at[dstb.at[b]], edb.at[b],
                              sem_ed.at[b]).wait()

    def _compute(b):
        @plsc.parallel_loop(0, CH, 1, unroll=4)
        def _(e):
            v = esb[b, e] + edb[b, e]
            v = jnp.where(v > 0, v, 0.01 * v)
            ex = jnp.exp(v - mvec)
            exb[b, e] = ex
            for g in range(8):
                zb[b, e, pl.ds(16 * g, 16)] = zb[b, e, pl.ds(16 * g, 16)] * ex

    def _start_scatters(b):
        pltpu.make_async_copy(exb.at[b], s_sh.at[dstb.at[b]],
                              sem_s.at[b]).start(add=True)
        pltpu.make_async_copy(zb.at[b], h_sh.at[dstb.at[b]],
                              sem_h.at[b]).start(add=True)

    def _wait_scatters(b):
        pltpu.make_async_copy(exb.at[b], s_sh.at[dstb.at[b]],
                              sem_s.at[b]).wait()
        pltpu.make_async_copy(zb.at[b], h_sh.at[dstb.at[b]],
                              sem_h.at[b]).wait()

    _start_gathers(0, 0)

    def pair_body(j, _):
        for b in range(2):
            i = 2 * j + b

            @pl.when(i > 0)
            def _():
                _wait_scatters(1 - b)

            @pl.when(i < NCHW - 2)
            def _():
                _start_gathers(1 - b, i + 1)

            _wait_gathers(b)
            _compute(b)
            _start_scatters(b)
        return 0

    lax.fori_loop(0, (NCHW - 1) // 2, pair_body, 0)
    _wait_scatters(1)

    _start_gathers(0, NCHW - 1)
    _wait_gathers(0)
    _compute(0)
    _start_scatters(0)
    _wait_scatters(0)

    plsc.subcore_barrier()

    @pl.when(sid < NS - 1)
    def _():
        pltpu.sync_copy(h_sh.at[pl.ds(row0, RSTEP), :],
                        h_out.at[cid, pl.ds(row0, RSTEP), :])
        pltpu.sync_copy(s_sh.at[pl.ds(row0, RSTEP), :],
                        s_out.at[cid, pl.ds(row0, RSTEP), :])

    @pl.when(sid == NS - 1)
    def _():
        pltpu.sync_copy(h_sh.at[pl.ds(row0, 400), :],
                        h_out.at[cid, pl.ds(row0, 400), :])
        pltpu.sync_copy(s_sh.at[pl.ds(row0, 400), :],
                        s_out.at[cid, pl.ds(row0, 400), :])


def _sc_edge_pass(zp, es, ed, m, src, dst):
    mesh = plsc.VectorSubcoreMesh(core_axis_name="c", subcore_axis_name="s")
    f = pl.kernel(
        _sc_body,
        out_type=(
            jax.ShapeDtypeStruct((NC, N, D), _f32),
            jax.ShapeDtypeStruct((NC, N, 16), _f32),
        ),
        mesh=mesh,
        scratch_types=[
            pltpu.VMEM_SHARED((N, D), _f32),
            pltpu.VMEM_SHARED((N, 16), _f32),
            pltpu.VMEM((2, CH), jnp.int32),
            pltpu.VMEM((2, CH), jnp.int32),
            pltpu.VMEM((2, CH, 16), _f32),
            pltpu.VMEM((2, CH, 16), _f32),
            pltpu.VMEM((2, CH, 16), _f32),
            pltpu.VMEM((2, CH, D), _f32),
            pltpu.VMEM((8, 16), _f32),
            pltpu.SemaphoreType.DMA((2,)),
            pltpu.SemaphoreType.DMA((2,)),
            pltpu.SemaphoreType.DMA((2,)),
            pltpu.SemaphoreType.DMA((2,)),
            pltpu.SemaphoreType.DMA((2,)),
        ],
        compiler_params=pltpu.CompilerParams(use_tc_tiling_on_sc=False),
    )
    return f(zp, es, ed, m, src, dst)


def _tc_post_body(hp_ref, sp_ref, t2_ref, p_ref, out_ref):
    ht = hp_ref[0] + hp_ref[1]
    st = sp_ref[0] + sp_ref[1]
    r2 = jnp.dot(1.0 / (st + 1e-16), t2_ref[...],
                 preferred_element_type=_f32,
                 precision=jax.lax.Precision.HIGHEST)
    scaled = (ht * r2).reshape(N, D)
    out_ref[...] = jnp.dot(scaled, p_ref[...],
                           preferred_element_type=_f32,
                           precision=jax.lax.Precision.HIGHEST)


def _tc_post(hp, sp, t2, perm):
    return pl.pallas_call(
        _tc_post_body,
        out_shape=jax.ShapeDtypeStruct((N, D), _f32),
    )(hp, sp, t2, perm)


@jax.jit
def kernel(x, edge_index, W, a):
    src = edge_index[0]
    dst = edge_index[1]
    wc = jnp.transpose(W, (1, 2, 0)).reshape(IN_DIM, D)
    a_src = a[:, :OUT_DIM, 0]
    a_dst = a[:, OUT_DIM:, 0]
    j = jnp.arange(D)
    heads = j % NUM_HEADS
    dims = j // NUM_HEADS
    hot = (heads[:, None] == jnp.arange(NUM_HEADS)[None, :]).astype(_f32)
    asrc_p = a_src.T[dims] * hot
    adst_p = a_dst.T[dims] * hot
    asrc_t = jnp.tile(asrc_p, (1, 4))
    adst_t = jnp.tile(adst_p, (1, 4))
    perm = jax.nn.one_hot(heads * OUT_DIM + dims, D, dtype=_f32)
    mcol = jnp.arange(8 * D)
    t2 = jax.nn.one_hot(16 * (mcol // D) + (mcol % D) % NUM_HEADS,
                        D, dtype=_f32).T

    zp, es, ed, m = _tc_pre(x, wc, asrc_t, adst_t)
    hp, sp = _sc_edge_pass(zp, es, ed, m, src, dst)
    hp8 = hp.reshape(NC, N // 8, 8 * D)
    sp8 = sp.reshape(NC, N // 8, 8 * 16)
    return _tc_post(hp8, sp8, t2, perm)


# speedup vs baseline: 103.6669x; 1.3734x over previous
import functools

import jax
import jax.numpy as jnp
from jax import lax
from jax.experimental import pallas as pl
from jax.experimental.pallas import tpu as pltpu
from jax.experimental.pallas import tpu_sc as plsc

N = 10000
E = 320000
IN_DIM = 128
OUT_DIM = 32
NUM_HEADS = 4
D = NUM_HEADS * OUT_DIM

NC = 2
NS = 16
NW = NC * NS
CH = 64
NCHUNK = E // CH
RSTEP = 640
NB = 3

_f32 = jnp.float32


def _tc_pre_body(x_ref, wc_ref, asrc_ref, adst_ref, zp_ref, es_ref, ed_ref,
                 m_ref):
    z = jnp.dot(x_ref[...], wc_ref[...], preferred_element_type=_f32)
    zp_ref[...] = z
    es = jnp.dot(z, asrc_ref[...], preferred_element_type=_f32)
    ed = jnp.dot(z, adst_ref[...], preferred_element_type=_f32)
    es_ref[...] = es
    ed_ref[...] = ed
    m = jnp.max(es, axis=0) + jnp.max(ed, axis=0)
    m = jnp.where(m > 0, m, 0.01 * m)
    m_ref[...] = jnp.broadcast_to(m[None, :], (8, 16))


def _tc_pre(x, wc, asrc_t, adst_t):
    return pl.pallas_call(
        _tc_pre_body,
        out_shape=(
            jax.ShapeDtypeStruct((N, D), _f32),
            jax.ShapeDtypeStruct((N, 16), _f32),
            jax.ShapeDtypeStruct((N, 16), _f32),
            jax.ShapeDtypeStruct((8, 16), _f32),
        ),
    )(x, wc, asrc_t, adst_t)


NCHW = NCHUNK // NW
NTAIL = NCHUNK - NW * NCHW


def _sc_body(zp_hbm, es_hbm, ed_hbm, m_hbm, ei_hbm,
             h_out, s_out,
             h_sh, s_sh,
             idxb, esb, edb, exb, zb, mb,
             sem_i, sem_es, sem_ed, sem_z, sem_s, sem_h):
    cid = lax.axis_index("c")
    sid = lax.axis_index("s")
    wid = sid * NC + cid
    zeros16 = jnp.zeros((16,), _f32)

    def zzh(i, _):
        for g in range(8):
            zb[0, i, pl.ds(16 * g, 16)] = zeros16
        esb[0, i] = zeros16
        return 0

    lax.fori_loop(0, CH, zzh, 0)
    row0 = sid * RSTEP

    def zcopy(r, _):
        pltpu.sync_copy(zb.at[0], h_sh.at[pl.ds(row0 + r * CH, CH), :])
        pltpu.sync_copy(esb.at[0], s_sh.at[pl.ds(row0 + r * CH, CH), :])
        return 0

    lax.fori_loop(0, jnp.where(sid == NS - 1, 6, 10), zcopy, 0)

    @pl.when(sid == NS - 1)
    def _():
        pltpu.sync_copy(zb.at[0, pl.ds(0, 16), :],
                        h_sh.at[pl.ds(row0 + 384, 16), :])
        pltpu.sync_copy(esb.at[0, pl.ds(0, 16), :],
                        s_sh.at[pl.ds(row0 + 384, 16), :])

    pltpu.sync_copy(m_hbm, mb)
    plsc.subcore_barrier()

    mvec = mb[0]

    def _start_idx(b, i):
        eb = (i * NW + wid) * CH
        pltpu.make_async_copy(ei_hbm.at[:, pl.ds(eb, CH)], idxb.at[b],
                              sem_i.at[b]).start()

    def _wait_idx(b, i):
        eb = (i * NW + wid) * CH
        pltpu.make_async_copy(ei_hbm.at[:, pl.ds(eb, CH)], idxb.at[b],
                              sem_i.at[b]).wait()

    def _start_gathers(b):
        pltpu.make_async_copy(zp_hbm.at[idxb.at[b, 0]], zb.at[b],
                              sem_z.at[b]).start()
        pltpu.make_async_copy(es_hbm.at[idxb.at[b, 0]], esb.at[b],
                              sem_es.at[b]).start()
        pltpu.make_async_copy(ed_hbm.at[idxb.at[b, 1]], edb.at[b],
                              sem_ed.at[b]).start()

    def _wait_gathers(b):
        pltpu.make_async_copy(zp_hbm.at[idxb.at[b, 0]], zb.at[b],
                              sem_z.at[b]).wait()
        pltpu.make_async_copy(es_hbm.at[idxb.at[b, 0]], esb.at[b],
                              sem_es.at[b]).wait()
        pltpu.make_async_copy(ed_hbm.at[idxb.at[b, 1]], edb.at[b],
                              sem_ed.at[b]).wait()

    def _compute(b):
        @plsc.parallel_loop(0, CH, 1, unroll=4)
        def _(e):
            v = esb[b, e] + edb[b, e]
            v = jnp.where(v > 0, v, 0.01 * v)
            ex = jnp.exp(v - mvec)
            exb[b, e] = ex
            for g in range(8):
                zb[b, e, pl.ds(16 * g, 16)] = zb[b, e, pl.ds(16 * g, 16)] * ex

    def _start_scatters(b):
        pltpu.make_async_copy(exb.at[b], s_sh.at[idxb.at[b, 1]],
                              sem_s.at[b]).start(add=True)
        pltpu.make_async_copy(zb.at[b], h_sh.at[idxb.at[b, 1]],
                              sem_h.at[b]).start(add=True)

    def _wait_scatters(b):
        pltpu.make_async_copy(exb.at[b], s_sh.at[idxb.at[b, 1]],
                              sem_s.at[b]).wait()
        pltpu.make_async_copy(zb.at[b], h_sh.at[idxb.at[b, 1]],
                              sem_h.at[b]).wait()

    _start_idx(0, 0)
    _start_idx(1, 1)
    _wait_idx(0, 0)
    _start_gathers(0)

    def trip_body(j, _):
        for b in range(NB):
            i = NB * j + b
            bm1 = (b - 1) % NB
            bp1 = (b + 1) % NB
            bp2 = (b + 2) % NB

            @pl.when(i > 0)
            def _():
                _wait_scatters(bm1)

            @pl.when(i + 2 < NCHW)
            def _():
                _start_idx(bp2, i + 2)

            @pl.when(i + 1 < NCHW)
            def _():
                _wait_idx(bp1, i + 1)
                _start_gathers(bp1)

            _wait_gathers(b)
            _compute(b)
            _start_scatters(b)
        return 0

    lax.fori_loop(0, NCHW // NB, trip_body, 0)
    _wait_scatters((NCHW - 1) % NB)

    @pl.when(wid < NTAIL)
    def _():
        _start_idx(0, NCHW)
        _wait_idx(0, NCHW)
        _start_gathers(0)
        _wait_gathers(0)
        _compute(0)
        _start_scatters(0)
        _wait_scatters(0)

    plsc.subcore_barrier()

    @pl.when(sid < NS - 1)
    def _():
        pltpu.sync_copy(h_sh.at[pl.ds(row0, RSTEP), :],
                        h_out.at[cid, pl.ds(row0, RSTEP), :])
        pltpu.sync_copy(s_sh.at[pl.ds(row0, RSTEP), :],
                        s_out.at[cid, pl.ds(row0, RSTEP), :])

    @pl.when(sid == NS - 1)
    def _():
        pltpu.sync_copy(h_sh.at[pl.ds(row0, 400), :],
                        h_out.at[cid, pl.ds(row0, 400), :])
        pltpu.sync_copy(s_sh.at[pl.ds(row0, 400), :],
                        s_out.at[cid, pl.ds(row0, 400), :])


def _sc_edge_pass(zp, es, ed, m, edge_index):
    mesh = plsc.VectorSubcoreMesh(core_axis_name="c", subcore_axis_name="s")
    f = pl.kernel(
        _sc_body,
        out_type=(
            jax.ShapeDtypeStruct((NC, N, D), _f32),
            jax.ShapeDtypeStruct((NC, N, 16), _f32),
        ),
        mesh=mesh,
        scratch_types=[
            pltpu.VMEM_SHARED((N, D), _f32),
            pltpu.VMEM_SHARED((N, 16), _f32),
            pltpu.VMEM((NB, 2, CH), jnp.int32),
            pltpu.VMEM((NB, CH, 16), _f32),
            pltpu.VMEM((NB, CH, 16), _f32),
            pltpu.VMEM((NB, CH, 16), _f32),
            pltpu.VMEM((NB, CH, D), _f32),
            pltpu.VMEM((8, 16), _f32),
            pltpu.SemaphoreType.DMA((NB,)),
            pltpu.SemaphoreType.DMA((NB,)),
            pltpu.SemaphoreType.DMA((NB,)),
            pltpu.SemaphoreType.DMA((NB,)),
            pltpu.SemaphoreType.DMA((NB,)),
            pltpu.SemaphoreType.DMA((NB,)),
        ],
        compiler_params=pltpu.CompilerParams(use_tc_tiling_on_sc=False),
    )
    return f(zp, es, ed, m, edge_index)


def _tc_post_body(hp_ref, sp_ref, t2_ref, p_ref, out_ref):
    ht = hp_ref[0] + hp_ref[1]
    st = sp_ref[0] + sp_ref[1]
    r2 = jnp.dot(1.0 / (st + 1e-16), t2_ref[...],
                 preferred_element_type=_f32,
                 precision=jax.lax.Precision.HIGHEST)
    scaled = (ht * r2).reshape(N, D)
    out_ref[...] = jnp.dot(scaled, p_ref[...],
                           preferred_element_type=_f32,
                           precision=jax.lax.Precision.HIGHEST)


def _tc_post(hp, sp, t2, perm):
    return pl.pallas_call(
        _tc_post_body,
        out_shape=jax.ShapeDtypeStruct((N, D), _f32),
    )(hp, sp, t2, perm)


@jax.jit
def kernel(x, edge_index, W, a):
    wc = jnp.transpose(W, (1, 2, 0)).reshape(IN_DIM, D)
    a_src = a[:, :OUT_DIM, 0]
    a_dst = a[:, OUT_DIM:, 0]
    j = jnp.arange(D)
    heads = j % NUM_HEADS
    dims = j // NUM_HEADS
    hot = (heads[:, None] == jnp.arange(NUM_HEADS)[None, :]).astype(_f32)
    asrc_p = a_src.T[dims] * hot
    adst_p = a_dst.T[dims] * hot
    asrc_t = jnp.tile(asrc_p, (1, 4))
    adst_t = jnp.tile(adst_p, (1, 4))
    perm = jax.nn.one_hot(heads * OUT_DIM + dims, D, dtype=_f32)
    mcol = jnp.arange(8 * D)
    t2 = jax.nn.one_hot(16 * (mcol // D) + (mcol % D) % NUM_HEADS,
                        D, dtype=_f32).T

    zp, es, ed, m = _tc_pre(x, wc, asrc_t, adst_t)
    hp, sp = _sc_edge_pass(zp, es, ed, m, edge_index)
    hp8 = hp.reshape(NC, N // 8, 8 * D)
    sp8 = sp.reshape(NC, N // 8, 8 * 16)
    return _tc_post(hp8, sp8, t2, perm)


# speedup vs baseline: 108.2278x; 1.0440x over previous
import functools

import jax
import jax.numpy as jnp
from jax import lax
from jax.experimental import pallas as pl
from jax.experimental.pallas import tpu as pltpu
from jax.experimental.pallas import tpu_sc as plsc

N = 10000
E = 320000
IN_DIM = 128
OUT_DIM = 32
NUM_HEADS = 4
D = NUM_HEADS * OUT_DIM

NC = 2
NS = 16
NW = NC * NS
CH = 80
NCHUNK = E // CH
RSTEP = 640
NB = 3

_f32 = jnp.float32


def _tc_pre_body(x_ref, wc_ref, asrc_ref, adst_ref, zp_ref, es_ref, ed_ref,
                 m_ref):
    z = jnp.dot(x_ref[...], wc_ref[...], preferred_element_type=_f32)
    zp_ref[...] = z
    es = jnp.dot(z, asrc_ref[...], preferred_element_type=_f32)
    ed = jnp.dot(z, adst_ref[...], preferred_element_type=_f32)
    es_ref[...] = es
    ed_ref[...] = ed
    m = jnp.max(es, axis=0) + jnp.max(ed, axis=0)
    m = jnp.where(m > 0, m, 0.01 * m)
    m_ref[...] = jnp.broadcast_to(m[None, :], (8, 16))


def _tc_pre(x, wc, asrc_t, adst_t):
    return pl.pallas_call(
        _tc_pre_body,
        out_shape=(
            jax.ShapeDtypeStruct((N, D), _f32),
            jax.ShapeDtypeStruct((N, 16), _f32),
            jax.ShapeDtypeStruct((N, 16), _f32),
            jax.ShapeDtypeStruct((8, 16), _f32),
        ),
    )(x, wc, asrc_t, adst_t)


NCHW = NCHUNK // NW


def _sc_body(zp_hbm, es_hbm, ed_hbm, m_hbm, ei_hbm,
             h_out, s_out,
             h_sh, s_sh,
             idxb, esb, edb, zb, mb,
             sem_i, sem_es, sem_ed, sem_z, sem_s, sem_h):
    cid = lax.axis_index("c")
    sid = lax.axis_index("s")
    wid = sid * NC + cid
    zeros16 = jnp.zeros((16,), _f32)

    def zzh(i, _):
        for g in range(8):
            zb[0, i, pl.ds(16 * g, 16)] = zeros16
        esb[0, i] = zeros16
        return 0

    lax.fori_loop(0, CH, zzh, 0)
    row0 = sid * RSTEP

    def zcopy(r, _):
        pltpu.sync_copy(zb.at[0], h_sh.at[pl.ds(row0 + r * CH, CH), :])
        pltpu.sync_copy(esb.at[0], s_sh.at[pl.ds(row0 + r * CH, CH), :])
        return 0

    lax.fori_loop(0, jnp.where(sid == NS - 1, 5, 8), zcopy, 0)
    pltpu.sync_copy(m_hbm, mb)
    plsc.subcore_barrier()

    mvec = mb[0]

    def _start_idx(b, i):
        eb = (i * NW + wid) * CH
        pltpu.make_async_copy(ei_hbm.at[:, pl.ds(eb, CH)], idxb.at[b],
                              sem_i.at[b]).start()

    def _wait_idx(b, i):
        eb = (i * NW + wid) * CH
        pltpu.make_async_copy(ei_hbm.at[:, pl.ds(eb, CH)], idxb.at[b],
                              sem_i.at[b]).wait()

    def _start_gathers(b):
        pltpu.make_async_copy(zp_hbm.at[idxb.at[b, 0]], zb.at[b],
                              sem_z.at[b]).start()
        pltpu.make_async_copy(es_hbm.at[idxb.at[b, 0]], esb.at[b],
                              sem_es.at[b]).start()
        pltpu.make_async_copy(ed_hbm.at[idxb.at[b, 1]], edb.at[b],
                              sem_ed.at[b]).start()

    def _wait_gathers(b):
        pltpu.make_async_copy(zp_hbm.at[idxb.at[b, 0]], zb.at[b],
                              sem_z.at[b]).wait()
        pltpu.make_async_copy(es_hbm.at[idxb.at[b, 0]], esb.at[b],
                              sem_es.at[b]).wait()
        pltpu.make_async_copy(ed_hbm.at[idxb.at[b, 1]], edb.at[b],
                              sem_ed.at[b]).wait()

    def _compute(b):
        @plsc.parallel_loop(0, CH, 1, unroll=4)
        def _(e):
            v = esb[b, e] + edb[b, e]
            v = jnp.where(v > 0, v, 0.01 * v)
            ex = jnp.exp(v - mvec)
            esb[b, e] = ex
            for g in range(8):
                zb[b, e, pl.ds(16 * g, 16)] = zb[b, e, pl.ds(16 * g, 16)] * ex

    def _start_scatters(b):
        pltpu.make_async_copy(esb.at[b], s_sh.at[idxb.at[b, 1]],
                              sem_s.at[b]).start(add=True)
        pltpu.make_async_copy(zb.at[b], h_sh.at[idxb.at[b, 1]],
                              sem_h.at[b]).start(add=True)

    def _wait_scatters(b):
        pltpu.make_async_copy(esb.at[b], s_sh.at[idxb.at[b, 1]],
                              sem_s.at[b]).wait()
        pltpu.make_async_copy(zb.at[b], h_sh.at[idxb.at[b, 1]],
                              sem_h.at[b]).wait()

    _start_idx(0, 0)
    _start_idx(1, 1)
    _wait_idx(0, 0)
    _start_gathers(0)

    def trip_body(j, _):
        for b in range(NB):
            i = NB * j + b
            bm1 = (b - 1) % NB
            bp1 = (b + 1) % NB
            bp2 = (b + 2) % NB

            @pl.when(i > 0)
            def _():
                _wait_scatters(bm1)

            @pl.when(i + 2 < NCHW)
            def _():
                _start_idx(bp2, i + 2)

            @pl.when(i + 1 < NCHW)
            def _():
                _wait_idx(bp1, i + 1)
                _start_gathers(bp1)

            _wait_gathers(b)
            _compute(b)
            _start_scatters(b)
        return 0

    lax.fori_loop(0, NCHW // NB, trip_body, 0)

    _wait_scatters(2)
    _wait_idx(1, NCHW - 1)
    _start_gathers(1)
    _wait_gathers(0)
    _compute(0)
    _start_scatters(0)
    _wait_scatters(0)
    _wait_gathers(1)
    _compute(1)
    _start_scatters(1)
    _wait_scatters(1)

    plsc.subcore_barrier()

    @pl.when(sid < NS - 1)
    def _():
        pltpu.sync_copy(h_sh.at[pl.ds(row0, RSTEP), :],
                        h_out.at[cid, pl.ds(row0, RSTEP), :])
        pltpu.sync_copy(s_sh.at[pl.ds(row0, RSTEP), :],
                        s_out.at[cid, pl.ds(row0, RSTEP), :])

    @pl.when(sid == NS - 1)
    def _():
        pltpu.sync_copy(h_sh.at[pl.ds(row0, 400), :],
                        h_out.at[cid, pl.ds(row0, 400), :])
        pltpu.sync_copy(s_sh.at[pl.ds(row0, 400), :],
                        s_out.at[cid, pl.ds(row0, 400), :])


def _sc_edge_pass(zp, es, ed, m, edge_index):
    mesh = plsc.VectorSubcoreMesh(core_axis_name="c", subcore_axis_name="s")
    f = pl.kernel(
        _sc_body,
        out_type=(
            jax.ShapeDtypeStruct((NC, N, D), _f32),
            jax.ShapeDtypeStruct((NC, N, 16), _f32),
        ),
        mesh=mesh,
        scratch_types=[
            pltpu.VMEM_SHARED((N, D), _f32),
            pltpu.VMEM_SHARED((N, 16), _f32),
            pltpu.VMEM((NB, 2, CH), jnp.int32),
            pltpu.VMEM((NB, CH, 16), _f32),
            pltpu.VMEM((NB, CH, 16), _f32),
            pltpu.VMEM((NB, CH, D), _f32),
            pltpu.VMEM((8, 16), _f32),
            pltpu.SemaphoreType.DMA((NB,)),
            pltpu.SemaphoreType.DMA((NB,)),
            pltpu.SemaphoreType.DMA((NB,)),
            pltpu.SemaphoreType.DMA((NB,)),
            pltpu.SemaphoreType.DMA((NB,)),
            pltpu.SemaphoreType.DMA((NB,)),
        ],
        compiler_params=pltpu.CompilerParams(use_tc_tiling_on_sc=False),
    )
    return f(zp, es, ed, m, edge_index)


def _tc_post_body(hp_ref, sp_ref, t2_ref, p_ref, out_ref):
    ht = hp_ref[0] + hp_ref[1]
    st = sp_ref[0] + sp_ref[1]
    r2 = jnp.dot(1.0 / (st + 1e-16), t2_ref[...],
                 preferred_element_type=_f32,
                 precision=jax.lax.Precision.HIGHEST)
    scaled = (ht * r2).reshape(N, D)
    out_ref[...] = jnp.dot(scaled, p_ref[...],
                           preferred_element_type=_f32,
                           precision=jax.lax.Precision.HIGHEST)


def _tc_post(hp, sp, t2, perm):
    return pl.pallas_call(
        _tc_post_body,
        out_shape=jax.ShapeDtypeStruct((N, D), _f32),
    )(hp, sp, t2, perm)


@jax.jit
def kernel(x, edge_index, W, a):
    wc = jnp.transpose(W, (1, 2, 0)).reshape(IN_DIM, D)
    a_src = a[:, :OUT_DIM, 0]
    a_dst = a[:, OUT_DIM:, 0]
    j = jnp.arange(D)
    heads = j % NUM_HEADS
    dims = j // NUM_HEADS
    hot = (heads[:, None] == jnp.arange(NUM_HEADS)[None, :]).astype(_f32)
    asrc_p = a_src.T[dims] * hot
    adst_p = a_dst.T[dims] * hot
    asrc_t = jnp.tile(asrc_p, (1, 4))
    adst_t = jnp.tile(adst_p, (1, 4))
    perm = jax.nn.one_hot(heads * OUT_DIM + dims, D, dtype=_f32)
    mcol = jnp.arange(8 * D)
    t2 = jax.nn.one_hot(16 * (mcol // D) + (mcol % D) % NUM_HEADS,
                        D, dtype=_f32).T

    zp, es, ed, m = _tc_pre(x, wc, asrc_t, adst_t)
    hp, sp = _sc_edge_pass(zp, es, ed, m, edge_index)
    hp8 = hp.reshape(NC, N // 8, 8 * D)
    sp8 = sp.reshape(NC, N // 8, 8 * 16)
    return _tc_post(hp8, sp8, t2, perm)


# speedup vs baseline: 108.3646x; 1.0013x over previous
import functools

import jax
import jax.numpy as jnp
from jax import lax
from jax.experimental import pallas as pl
from jax.experimental.pallas import tpu as pltpu
from jax.experimental.pallas import tpu_sc as plsc

N = 10000
E = 320000
IN_DIM = 128
OUT_DIM = 32
NUM_HEADS = 4
D = NUM_HEADS * OUT_DIM

NC = 2
NS = 16
NW = NC * NS
CH = 80
NCHUNK = E // CH
RSTEP = 640
NB = 3

_f32 = jnp.float32


_HI = jax.lax.Precision.HIGHEST


def _iota(shape, dim):
    return lax.broadcasted_iota(jnp.int32, shape, dim)


def _tc_pre_body(x_ref, w_ref, a_ref, zp_ref, es_ref, ed_ref, m_ref):
    d32 = _iota((OUT_DIM, D), 0)
    j128 = _iota((OUT_DIM, D), 1)
    wc = jnp.dot(w_ref[0], (j128 == 4 * d32).astype(_f32),
                 preferred_element_type=_f32, precision=_HI)
    for h in range(1, NUM_HEADS):
        wc = wc + jnp.dot(w_ref[h], (j128 == 4 * d32 + h).astype(_f32),
                          preferred_element_type=_f32, precision=_HI)
    z = jnp.dot(x_ref[...], wc, preferred_element_type=_f32)
    zp_ref[...] = z

    a_mat = a_ref[...].reshape(NUM_HEADS, 2 * OUT_DIM)
    jj = _iota((D, 2 * OUT_DIM), 0)
    dd = _iota((D, 2 * OUT_DIM), 1)
    l_src = (dd == jj // 4).astype(_f32)
    l_dst = (dd == OUT_DIM + jj // 4).astype(_f32)
    dn = (((1,), (1,)), ((), ()))
    p_src = lax.dot_general(l_src, a_mat, dn, precision=_HI)
    p_dst = lax.dot_general(l_dst, a_mat, dn, precision=_HI)
    r4 = (_iota((NUM_HEADS, 16), 0) == _iota((NUM_HEADS, 16), 1) % 4)
    mask = (_iota((D, 16), 0) % 4 == _iota((D, 16), 1) % 4).astype(_f32)
    asrc_t = jnp.dot(p_src, r4.astype(_f32), precision=_HI) * mask
    adst_t = jnp.dot(p_dst, r4.astype(_f32), precision=_HI) * mask

    es = jnp.dot(z, asrc_t, preferred_element_type=_f32)
    ed = jnp.dot(z, adst_t, preferred_element_type=_f32)
    es_ref[...] = es
    ed_ref[...] = ed
    m = jnp.max(es, axis=0) + jnp.max(ed, axis=0)
    m = jnp.where(m > 0, m, 0.01 * m)
    m_ref[...] = jnp.broadcast_to(m[None, :], (8, 16))


def _tc_pre(x, w, a):
    return pl.pallas_call(
        _tc_pre_body,
        out_shape=(
            jax.ShapeDtypeStruct((N, D), _f32),
            jax.ShapeDtypeStruct((N, 16), _f32),
            jax.ShapeDtypeStruct((N, 16), _f32),
            jax.ShapeDtypeStruct((8, 16), _f32),
        ),
    )(x, w, a)


NCHW = NCHUNK // NW


def _sc_body(zp_hbm, es_hbm, ed_hbm, m_hbm, ei_hbm,
             h_out, s_out,
             h_sh, s_sh,
             idxb, esb, edb, zb, mb,
             sem_i, sem_es, sem_ed, sem_z, sem_s, sem_h):
    cid = lax.axis_index("c")
    sid = lax.axis_index("s")
    wid = sid * NC + cid
    zeros16 = jnp.zeros((16,), _f32)

    def zzh(i, _):
        for g in range(8):
            zb[0, i, pl.ds(16 * g, 16)] = zeros16
        esb[0, i] = zeros16
        return 0

    lax.fori_loop(0, CH, zzh, 0)
    row0 = sid * RSTEP

    def zcopy(r, _):
        pltpu.sync_copy(zb.at[0], h_sh.at[pl.ds(row0 + r * CH, CH), :])
        pltpu.sync_copy(esb.at[0], s_sh.at[pl.ds(row0 + r * CH, CH), :])
        return 0

    lax.fori_loop(0, jnp.where(sid == NS - 1, 5, 8), zcopy, 0)
    pltpu.sync_copy(m_hbm, mb)
    plsc.subcore_barrier()

    mvec = mb[0]

    def _start_idx(b, i):
        eb = (i * NW + wid) * CH
        pltpu.make_async_copy(ei_hbm.at[:, pl.ds(eb, CH)], idxb.at[b],
                              sem_i.at[b]).start()

    def _wait_idx(b, i):
        eb = (i * NW + wid) * CH
        pltpu.make_async_copy(ei_hbm.at[:, pl.ds(eb, CH)], idxb.at[b],
                              sem_i.at[b]).wait()

    def _start_gathers(b):
        pltpu.make_async_copy(zp_hbm.at[idxb.at[b, 0]], zb.at[b],
                              sem_z.at[b]).start()
        pltpu.make_async_copy(es_hbm.at[idxb.at[b, 0]], esb.at[b],
                              sem_es.at[b]).start()
        pltpu.make_async_copy(ed_hbm.at[idxb.at[b, 1]], edb.at[b],
                              sem_ed.at[b]).start()

    def _wait_gathers(b):
        pltpu.make_async_copy(zp_hbm.at[idxb.at[b, 0]], zb.at[b],
                              sem_z.at[b]).wait()
        pltpu.make_async_copy(es_hbm.at[idxb.at[b, 0]], esb.at[b],
                              sem_es.at[b]).wait()
        pltpu.make_async_copy(ed_hbm.at[idxb.at[b, 1]], edb.at[b],
                              sem_ed.at[b]).wait()

    def _compute(b):
        @plsc.parallel_loop(0, CH, 1, unroll=4)
        def _(e):
            v = esb[b, e] + edb[b, e]
            v = jnp.where(v > 0, v, 0.01 * v)
            ex = jnp.exp(v - mvec)
            esb[b, e] = ex
            for g in range(8):
                zb[b, e, pl.ds(16 * g, 16)] = zb[b, e, pl.ds(16 * g, 16)] * ex

    def _start_scatters(b):
        pltpu.make_async_copy(esb.at[b], s_sh.at[idxb.at[b, 1]],
                              sem_s.at[b]).start(add=True)
        pltpu.make_async_copy(zb.at[b], h_sh.at[idxb.at[b, 1]],
                              sem_h.at[b]).start(add=True)

    def _wait_scatters(b):
        pltpu.make_async_copy(esb.at[b], s_sh.at[idxb.at[b, 1]],
                              sem_s.at[b]).wait()
        pltpu.make_async_copy(zb.at[b], h_sh.at[idxb.at[b, 1]],
                              sem_h.at[b]).wait()

    _start_idx(0, 0)
    _start_idx(1, 1)
    _wait_idx(0, 0)
    _start_gathers(0)

    def trip_body(j, _):
        for b in range(NB):
            i = NB * j + b
            bm1 = (b - 1) % NB
            bp1 = (b + 1) % NB
            bp2 = (b + 2) % NB

            @pl.when(i > 0)
            def _():
                _wait_scatters(bm1)

            @pl.when(i + 2 < NCHW)
            def _():
                _start_idx(bp2, i + 2)

            @pl.when(i + 1 < NCHW)
            def _():
                _wait_idx(bp1, i + 1)
                _start_gathers(bp1)

            _wait_gathers(b)
            _compute(b)
            _start_scatters(b)
        return 0

    lax.fori_loop(0, NCHW // NB, trip_body, 0)

    _wait_scatters(2)
    _wait_idx(1, NCHW - 1)
    _start_gathers(1)
    _wait_gathers(0)
    _compute(0)
    _start_scatters(0)
    _wait_scatters(0)
    _wait_gathers(1)
    _compute(1)
    _start_scatters(1)
    _wait_scatters(1)

    plsc.subcore_barrier()

    @pl.when(sid < NS - 1)
    def _():
        pltpu.sync_copy(h_sh.at[pl.ds(row0, RSTEP), :],
                        h_out.at[cid, pl.ds(row0, RSTEP), :])
        pltpu.sync_copy(s_sh.at[pl.ds(row0, RSTEP), :],
                        s_out.at[cid, pl.ds(row0, RSTEP), :])

    @pl.when(sid == NS - 1)
    def _():
        pltpu.sync_copy(h_sh.at[pl.ds(row0, 400), :],
                        h_out.at[cid, pl.ds(row0, 400), :])
        pltpu.sync_copy(s_sh.at[pl.ds(row0, 400), :],
                        s_out.at[cid, pl.ds(row0, 400), :])


def _sc_edge_pass(zp, es, ed, m, edge_index):
    mesh = plsc.VectorSubcoreMesh(core_axis_name="c", subcore_axis_name="s")
    f = pl.kernel(
        _sc_body,
        out_type=(
            jax.ShapeDtypeStruct((NC, N, D), _f32),
            jax.ShapeDtypeStruct((NC, N, 16), _f32),
        ),
        mesh=mesh,
        scratch_types=[
            pltpu.VMEM_SHARED((N, D), _f32),
            pltpu.VMEM_SHARED((N, 16), _f32),
            pltpu.VMEM((NB, 2, CH), jnp.int32),
            pltpu.VMEM((NB, CH, 16), _f32),
            pltpu.VMEM((NB, CH, 16), _f32),
            pltpu.VMEM((NB, CH, D), _f32),
            pltpu.VMEM((8, 16), _f32),
            pltpu.SemaphoreType.DMA((NB,)),
            pltpu.SemaphoreType.DMA((NB,)),
            pltpu.SemaphoreType.DMA((NB,)),
            pltpu.SemaphoreType.DMA((NB,)),
            pltpu.SemaphoreType.DMA((NB,)),
            pltpu.SemaphoreType.DMA((NB,)),
        ],
        compiler_params=pltpu.CompilerParams(use_tc_tiling_on_sc=False),
    )
    return f(zp, es, ed, m, edge_index)


def _tc_post_body(hp_ref, sp_ref, t2_ref, p_ref, out_ref):
    ht = hp_ref[0] + hp_ref[1]
    st = sp_ref[0] + sp_ref[1]
    r2 = jnp.dot(1.0 / (st + 1e-16), t2_ref[...],
                 preferred_element_type=_f32,
                 precision=jax.lax.Precision.HIGHEST)
    scaled = (ht * r2).reshape(N, D)
    out_ref[...] = jnp.dot(scaled, p_ref[...],
                           preferred_element_type=_f32,
                           precision=jax.lax.Precision.HIGHEST)


def _tc_post(hp, sp, t2, perm):
    return pl.pallas_call(
        _tc_post_body,
        out_shape=jax.ShapeDtypeStruct((N, D), _f32),
    )(hp, sp, t2, perm)


@jax.jit
def kernel(x, edge_index, W, a):
    j = jnp.arange(D)
    heads = j % NUM_HEADS
    dims = j // NUM_HEADS
    perm = jax.nn.one_hot(heads * OUT_DIM + dims, D, dtype=_f32)
    mcol = jnp.arange(8 * D)
    t2 = jax.nn.one_hot(16 * (mcol // D) + (mcol % D) % NUM_HEADS,
                        D, dtype=_f32).T

    zp, es, ed, m = _tc_pre(x, W, a)
    hp, sp = _sc_edge_pass(zp, es, ed, m, edge_index)
    hp8 = hp.reshape(NC, N // 8, 8 * D)
    sp8 = sp.reshape(NC, N // 8, 8 * 16)
    return _tc_post(hp8, sp8, t2, perm)


# speedup vs baseline: 108.8903x; 1.0049x over previous
import functools

import jax
import jax.numpy as jnp
from jax import lax
from jax.experimental import pallas as pl
from jax.experimental.pallas import tpu as pltpu
from jax.experimental.pallas import tpu_sc as plsc

N = 10000
E = 320000
IN_DIM = 128
OUT_DIM = 32
NUM_HEADS = 4
D = NUM_HEADS * OUT_DIM

NC = 2
NS = 16
NW = NC * NS
CH = 80
NCHUNK = E // CH
RSTEP = 640
NB = 3

_f32 = jnp.float32


_HI = jax.lax.Precision.HIGHEST


def _iota(shape, dim):
    return lax.broadcasted_iota(jnp.int32, shape, dim)


def _tc_pre_body(x_ref, w_ref, a_ref, zp_ref, es_ref, ed_ref, m_ref):
    d32 = _iota((OUT_DIM, D), 0)
    j128 = _iota((OUT_DIM, D), 1)
    wc = jnp.dot(w_ref[0], (j128 == 4 * d32).astype(_f32),
                 preferred_element_type=_f32, precision=_HI)
    for h in range(1, NUM_HEADS):
        wc = wc + jnp.dot(w_ref[h], (j128 == 4 * d32 + h).astype(_f32),
                          preferred_element_type=_f32, precision=_HI)
    z = jnp.dot(x_ref[...], wc, preferred_element_type=_f32)
    zp_ref[...] = z

    a_mat = a_ref[...].reshape(NUM_HEADS, 2 * OUT_DIM)
    jj = _iota((D, 2 * OUT_DIM), 0)
    dd = _iota((D, 2 * OUT_DIM), 1)
    l_src = (dd == jj // 4).astype(_f32)
    l_dst = (dd == OUT_DIM + jj // 4).astype(_f32)
    dn = (((1,), (1,)), ((), ()))
    p_src = lax.dot_general(l_src, a_mat, dn, precision=_HI)
    p_dst = lax.dot_general(l_dst, a_mat, dn, precision=_HI)
    r4 = (_iota((NUM_HEADS, 16), 0) == _iota((NUM_HEADS, 16), 1) % 4)
    mask = (_iota((D, 16), 0) % 4 == _iota((D, 16), 1) % 4).astype(_f32)
    asrc_t = jnp.dot(p_src, r4.astype(_f32), precision=_HI) * mask
    adst_t = jnp.dot(p_dst, r4.astype(_f32), precision=_HI) * mask

    es = jnp.dot(z, asrc_t, preferred_element_type=_f32)
    ed = jnp.dot(z, adst_t, preferred_element_type=_f32)
    es_ref[...] = es
    ed_ref[...] = ed
    m = jnp.max(es, axis=0) + jnp.max(ed, axis=0)
    m = jnp.where(m > 0, m, 0.01 * m)
    m_ref[...] = jnp.broadcast_to(m[None, :], (8, 16))


def _tc_pre(x, w, a):
    return pl.pallas_call(
        _tc_pre_body,
        out_shape=(
            jax.ShapeDtypeStruct((N, D), _f32),
            jax.ShapeDtypeStruct((N, 16), _f32),
            jax.ShapeDtypeStruct((N, 16), _f32),
            jax.ShapeDtypeStruct((8, 16), _f32),
        ),
    )(x, w, a)


NCHW = NCHUNK // NW


def _sc_body(zp_hbm, es_hbm, ed_hbm, m_hbm, ei_hbm,
             h_out, s_out,
             h_sh, s_sh,
             idxb, esb, edb, zb, mb,
             sem_i, sem_es, sem_ed, sem_z, sem_s, sem_h):
    cid = lax.axis_index("c")
    sid = lax.axis_index("s")
    wid = sid * NC + cid
    zeros16 = jnp.zeros((16,), _f32)

    def zzh(i, _):
        for g in range(8):
            zb[0, i, pl.ds(16 * g, 16)] = zeros16
        esb[0, i] = zeros16
        return 0

    lax.fori_loop(0, CH, zzh, 0)
    row0 = sid * RSTEP

    nz = jnp.where(sid == NS - 1, 5, 8)

    def zcopy(r, _):
        pltpu.make_async_copy(zb.at[0], h_sh.at[pl.ds(row0 + r * CH, CH), :],
                              sem_h.at[0]).start()
        pltpu.make_async_copy(esb.at[0], s_sh.at[pl.ds(row0 + r * CH, CH), :],
                              sem_s.at[0]).start()
        return 0

    def zwait(r, _):
        pltpu.make_async_copy(zb.at[0], h_sh.at[pl.ds(row0 + r * CH, CH), :],
                              sem_h.at[0]).wait()
        pltpu.make_async_copy(esb.at[0], s_sh.at[pl.ds(row0 + r * CH, CH), :],
                              sem_s.at[0]).wait()
        return 0

    lax.fori_loop(0, nz, zcopy, 0)
    lax.fori_loop(0, nz, zwait, 0)
    pltpu.sync_copy(m_hbm, mb)
    plsc.subcore_barrier()

    mvec = mb[0]

    def _start_idx(b, i):
        eb = (i * NW + wid) * CH
        pltpu.make_async_copy(ei_hbm.at[:, pl.ds(eb, CH)], idxb.at[b],
                              sem_i.at[b]).start()

    def _wait_idx(b, i):
        eb = (i * NW + wid) * CH
        pltpu.make_async_copy(ei_hbm.at[:, pl.ds(eb, CH)], idxb.at[b],
                              sem_i.at[b]).wait()

    def _start_gathers(b):
        pltpu.make_async_copy(zp_hbm.at[idxb.at[b, 0]], zb.at[b],
                              sem_z.at[b]).start()
        pltpu.make_async_copy(es_hbm.at[idxb.at[b, 0]], esb.at[b],
                              sem_es.at[b]).start()
        pltpu.make_async_copy(ed_hbm.at[idxb.at[b, 1]], edb.at[b],
                              sem_ed.at[b]).start()

    def _wait_gathers(b):
        pltpu.make_async_copy(zp_hbm.at[idxb.at[b, 0]], zb.at[b],
                              sem_z.at[b]).wait()
        pltpu.make_async_copy(es_hbm.at[idxb.at[b, 0]], esb.at[b],
                              sem_es.at[b]).wait()
        pltpu.make_async_copy(ed_hbm.at[idxb.at[b, 1]], edb.at[b],
                              sem_ed.at[b]).wait()

    def _compute(b):
        @plsc.parallel_loop(0, CH, 1, unroll=8)
        def _(e):
            v = esb[b, e] + edb[b, e]
            v = jnp.where(v > 0, v, 0.01 * v)
            ex = jnp.exp(v - mvec)
            esb[b, e] = ex
            for g in range(8):
                zb[b, e, pl.ds(16 * g, 16)] = zb[b, e, pl.ds(16 * g, 16)] * ex

    def _start_scatters(b):
        pltpu.make_async_copy(esb.at[b], s_sh.at[idxb.at[b, 1]],
                              sem_s.at[b]).start(add=True)
        pltpu.make_async_copy(zb.at[b], h_sh.at[idxb.at[b, 1]],
                              sem_h.at[b]).start(add=True)

    def _wait_scatters(b):
        pltpu.make_async_copy(esb.at[b], s_sh.at[idxb.at[b, 1]],
                              sem_s.at[b]).wait()
        pltpu.make_async_copy(zb.at[b], h_sh.at[idxb.at[b, 1]],
                              sem_h.at[b]).wait()

    _start_idx(0, 0)
    _start_idx(1, 1)
    _wait_idx(0, 0)
    _start_gathers(0)

    def trip_body(j, _):
        for b in range(NB):
            i = NB * j + b
            bm1 = (b - 1) % NB
            bp1 = (b + 1) % NB
            bp2 = (b + 2) % NB

            @pl.when(i > 0)
            def _():
                _wait_scatters(bm1)

            @pl.when(i + 2 < NCHW)
            def _():
                _start_idx(bp2, i + 2)

            @pl.when(i + 1 < NCHW)
            def _():
                _wait_idx(bp1, i + 1)
                _start_gathers(bp1)

            _wait_gathers(b)
            _compute(b)
            _start_scatters(b)
        return 0

    lax.fori_loop(0, NCHW // NB, trip_body, 0)

    _wait_scatters(2)
    _wait_idx(1, NCHW - 1)
    _start_gathers(1)
    _wait_gathers(0)
    _compute(0)
    _start_scatters(0)
    _wait_scatters(0)
    _wait_gathers(1)
    _compute(1)
    _start_scatters(1)
    _wait_scatters(1)

    plsc.subcore_barrier()

    @pl.when(sid < NS - 1)
    def _():
        pltpu.make_async_copy(h_sh.at[pl.ds(row0, RSTEP), :],
                              h_out.at[cid, pl.ds(row0, RSTEP), :],
                              sem_h.at[0]).start()
        pltpu.make_async_copy(s_sh.at[pl.ds(row0, RSTEP), :],
                              s_out.at[cid, pl.ds(row0, RSTEP), :],
                              sem_s.at[0]).start()
        pltpu.make_async_copy(h_sh.at[pl.ds(row0, RSTEP), :],
                              h_out.at[cid, pl.ds(row0, RSTEP), :],
                              sem_h.at[0]).wait()
        pltpu.make_async_copy(s_sh.at[pl.ds(row0, RSTEP), :],
                              s_out.at[cid, pl.ds(row0, RSTEP), :],
                              sem_s.at[0]).wait()

    @pl.when(sid == NS - 1)
    def _():
        pltpu.make_async_copy(h_sh.at[pl.ds(row0, 400), :],
                              h_out.at[cid, pl.ds(row0, 400), :],
                              sem_h.at[0]).start()
        pltpu.make_async_copy(s_sh.at[pl.ds(row0, 400), :],
                              s_out.at[cid, pl.ds(row0, 400), :],
                              sem_s.at[0]).start()
        pltpu.make_async_copy(h_sh.at[pl.ds(row0, 400), :],
                              h_out.at[cid, pl.ds(row0, 400), :],
                              sem_h.at[0]).wait()
        pltpu.make_async_copy(s_sh.at[pl.ds(row0, 400), :],
                              s_out.at[cid, pl.ds(row0, 400), :],
                              sem_s.at[0]).wait()


def _sc_edge_pass(zp, es, ed, m, edge_index):
    mesh = plsc.VectorSubcoreMesh(core_axis_name="c", subcore_axis_name="s")
    f = pl.kernel(
        _sc_body,
        out_type=(
            jax.ShapeDtypeStruct((NC, N, D), _f32),
            jax.ShapeDtypeStruct((NC, N, 16), _f32),
        ),
        mesh=mesh,
        scratch_types=[
            pltpu.VMEM_SHARED((N, D), _f32),
            pltpu.VMEM_SHARED((N, 16), _f32),
            pltpu.VMEM((NB, 2, CH), jnp.int32),
            pltpu.VMEM((NB, CH, 16), _f32),
            pltpu.VMEM((NB, CH, 16), _f32),
            pltpu.VMEM((NB, CH, D), _f32),
            pltpu.VMEM((8, 16), _f32),
            pltpu.SemaphoreType.DMA((NB,)),
            pltpu.SemaphoreType.DMA((NB,)),
            pltpu.SemaphoreType.DMA((NB,)),
            pltpu.SemaphoreType.DMA((NB,)),
            pltpu.SemaphoreType.DMA((NB,)),
            pltpu.SemaphoreType.DMA((NB,)),
        ],
        compiler_params=pltpu.CompilerParams(use_tc_tiling_on_sc=False),
    )
    return f(zp, es, ed, m, edge_index)


def _tc_post_body(hp_ref, sp_ref, t2_ref, p_ref, out_ref):
    ht = hp_ref[0] + hp_ref[1]
    st = sp_ref[0] + sp_ref[1]
    r2 = jnp.dot(1.0 / (st + 1e-16), t2_ref[...],
                 preferred_element_type=_f32,
                 precision=jax.lax.Precision.HIGHEST)
    scaled = (ht * r2).reshape(N, D)
    out_ref[...] = jnp.dot(scaled, p_ref[...],
                           preferred_element_type=_f32,
                           precision=jax.lax.Precision.HIGHEST)


def _tc_post(hp, sp, t2, perm):
    return pl.pallas_call(
        _tc_post_body,
        out_shape=jax.ShapeDtypeStruct((N, D), _f32),
    )(hp, sp, t2, perm)


@jax.jit
def kernel(x, edge_index, W, a):
    j = jnp.arange(D)
    heads = j % NUM_HEADS
    dims = j // NUM_HEADS
    perm = jax.nn.one_hot(heads * OUT_DIM + dims, D, dtype=_f32)
    mcol = jnp.arange(8 * D)
    t2 = jax.nn.one_hot(16 * (mcol // D) + (mcol % D) % NUM_HEADS,
                        D, dtype=_f32).T

    zp, es, ed, m = _tc_pre(x, W, a)
    hp, sp = _sc_edge_pass(zp, es, ed, m, edge_index)
    hp8 = hp.reshape(NC, N // 8, 8 * D)
    sp8 = sp.reshape(NC, N // 8, 8 * 16)
    return _tc_post(hp8, sp8, t2, perm)


# speedup vs baseline: 109.0425x; 1.0014x over previous
import functools

import jax
import jax.numpy as jnp
from jax import lax
from jax.experimental import pallas as pl
from jax.experimental.pallas import tpu as pltpu
from jax.experimental.pallas import tpu_sc as plsc

N = 10000
E = 320000
IN_DIM = 128
OUT_DIM = 32
NUM_HEADS = 4
D = NUM_HEADS * OUT_DIM

NC = 2
NS = 16
NW = NC * NS
CH = 80
NCHUNK = E // CH
RSTEP = 640
NB = 3

_f32 = jnp.float32


_HI = jax.lax.Precision.HIGHEST


def _iota(shape, dim):
    return lax.broadcasted_iota(jnp.int32, shape, dim)


def _tc_pre_body(x_ref, w_ref, a_ref, zp_ref, es_ref, ed_ref, m_ref):
    d32 = _iota((OUT_DIM, D), 0)
    j128 = _iota((OUT_DIM, D), 1)
    wc = jnp.dot(w_ref[0], (j128 == 4 * d32).astype(_f32),
                 preferred_element_type=_f32, precision=_HI)
    for h in range(1, NUM_HEADS):
        wc = wc + jnp.dot(w_ref[h], (j128 == 4 * d32 + h).astype(_f32),
                          preferred_element_type=_f32, precision=_HI)
    z = jnp.dot(x_ref[...], wc, preferred_element_type=_f32)
    zp_ref[...] = z

    a_mat = a_ref[...].reshape(NUM_HEADS, 2 * OUT_DIM)
    jj = _iota((D, 2 * OUT_DIM), 0)
    dd = _iota((D, 2 * OUT_DIM), 1)
    l_src = (dd == jj // 4).astype(_f32)
    l_dst = (dd == OUT_DIM + jj // 4).astype(_f32)
    dn = (((1,), (1,)), ((), ()))
    p_src = lax.dot_general(l_src, a_mat, dn, precision=_HI)
    p_dst = lax.dot_general(l_dst, a_mat, dn, precision=_HI)
    r4 = (_iota((NUM_HEADS, 16), 0) == _iota((NUM_HEADS, 16), 1) % 4)
    mask = (_iota((D, 16), 0) % 4 == _iota((D, 16), 1) % 4).astype(_f32)
    asrc_t = jnp.dot(p_src, r4.astype(_f32), precision=_HI) * mask
    adst_t = jnp.dot(p_dst, r4.astype(_f32), precision=_HI) * mask

    es = jnp.dot(z, asrc_t, preferred_element_type=_f32)
    ed = jnp.dot(z, adst_t, preferred_element_type=_f32)
    es_ref[...] = es
    ed_ref[...] = ed
    m = jnp.max(es, axis=0) + jnp.max(ed, axis=0)
    m = jnp.where(m > 0, m, 0.01 * m)
    m_ref[...] = jnp.broadcast_to(m[None, :], (8, 16))


def _tc_pre(x, w, a):
    return pl.pallas_call(
        _tc_pre_body,
        out_shape=(
            jax.ShapeDtypeStruct((N, D), _f32),
            jax.ShapeDtypeStruct((N, 16), _f32),
            jax.ShapeDtypeStruct((N, 16), _f32),
            jax.ShapeDtypeStruct((8, 16), _f32),
        ),
    )(x, w, a)


NCHW = NCHUNK // NW


def _sc_body(zp_hbm, es_hbm, ed_hbm, m_hbm, ei_hbm,
             h_out, s_out,
             h_sh, s_sh,
             idxb, esb, edb, zb, mb,
             sem_i, sem_es, sem_ed, sem_z, sem_s, sem_h):
    cid = lax.axis_index("c")
    sid = lax.axis_index("s")
    wid = sid * NC + cid
    zeros16 = jnp.zeros((16,), _f32)

    def zzh(i, _):
        for g in range(8):
            zb[0, i, pl.ds(16 * g, 16)] = zeros16
        esb[0, i] = zeros16
        return 0

    lax.fori_loop(0, CH, zzh, 0)
    row0 = sid * RSTEP

    nz = jnp.where(sid == NS - 1, 5, 8)

    def zcopy(r, _):
        pltpu.make_async_copy(zb.at[0], h_sh.at[pl.ds(row0 + r * CH, CH), :],
                              sem_h.at[0]).start()
        pltpu.make_async_copy(esb.at[0], s_sh.at[pl.ds(row0 + r * CH, CH), :],
                              sem_s.at[0]).start()
        return 0

    def zwait(r, _):
        pltpu.make_async_copy(zb.at[0], h_sh.at[pl.ds(row0 + r * CH, CH), :],
                              sem_h.at[0]).wait()
        pltpu.make_async_copy(esb.at[0], s_sh.at[pl.ds(row0 + r * CH, CH), :],
                              sem_s.at[0]).wait()
        return 0

    lax.fori_loop(0, nz, zcopy, 0)
    lax.fori_loop(0, nz, zwait, 0)
    pltpu.sync_copy(m_hbm, mb)
    plsc.subcore_barrier()

    mvec = mb[0]

    def _start_idx(b, i):
        eb = (i * NW + wid) * CH
        pltpu.make_async_copy(ei_hbm.at[:, pl.ds(eb, CH)], idxb.at[b],
                              sem_i.at[b]).start()

    def _wait_idx(b, i):
        eb = (i * NW + wid) * CH
        pltpu.make_async_copy(ei_hbm.at[:, pl.ds(eb, CH)], idxb.at[b],
                              sem_i.at[b]).wait()

    def _start_gathers(b):
        pltpu.make_async_copy(zp_hbm.at[idxb.at[b, 0]], zb.at[b],
                              sem_z.at[b]).start()
        pltpu.make_async_copy(es_hbm.at[idxb.at[b, 0]], esb.at[b],
                              sem_es.at[b]).start()
        pltpu.make_async_copy(ed_hbm.at[idxb.at[b, 1]], edb.at[b],
                              sem_ed.at[b]).start()

    def _wait_gathers(b):
        pltpu.make_async_copy(zp_hbm.at[idxb.at[b, 0]], zb.at[b],
                              sem_z.at[b]).wait()
        pltpu.make_async_copy(es_hbm.at[idxb.at[b, 0]], esb.at[b],
                              sem_es.at[b]).wait()
        pltpu.make_async_copy(ed_hbm.at[idxb.at[b, 1]], edb.at[b],
                              sem_ed.at[b]).wait()

    def _compute(b):
        @plsc.parallel_loop(0, CH, 1, unroll=8)
        def _(e):
            v = esb[b, e] + edb[b, e]
            v = jnp.where(v > 0, v, 0.01 * v)
            ex = jnp.exp(v - mvec)
            esb[b, e] = ex
            for g in range(8):
                zb[b, e, pl.ds(16 * g, 16)] = zb[b, e, pl.ds(16 * g, 16)] * ex

    def _start_scatters(b):
        pltpu.make_async_copy(esb.at[b], s_sh.at[idxb.at[b, 1]],
                              sem_s.at[b]).start(add=True)
        pltpu.make_async_copy(zb.at[b], h_sh.at[idxb.at[b, 1]],
                              sem_h.at[b]).start(add=True)

    def _wait_scatters(b):
        pltpu.make_async_copy(esb.at[b], s_sh.at[idxb.at[b, 1]],
                              sem_s.at[b]).wait()
        pltpu.make_async_copy(zb.at[b], h_sh.at[idxb.at[b, 1]],
                              sem_h.at[b]).wait()

    _start_idx(0, 0)
    _start_idx(1, 1)
    _wait_idx(0, 0)
    _start_gathers(0)

    def trip_body(j, _):
        for b in range(NB):
            i = NB * j + b
            bm1 = (b - 1) % NB
            bp1 = (b + 1) % NB
            bp2 = (b + 2) % NB

            @pl.when(i > 0)
            def _():
                _wait_scatters(bm1)

            @pl.when(i + 2 < NCHW)
            def _():
                _start_idx(bp2, i + 2)

            @pl.when(i + 1 < NCHW)
            def _():
                _wait_idx(bp1, i + 1)
                _start_gathers(bp1)

            _wait_gathers(b)
            _compute(b)
            _start_scatters(b)
        return 0

    lax.fori_loop(0, NCHW // NB, trip_body, 0)

    _wait_scatters(2)
    _wait_idx(1, NCHW - 1)
    _start_gathers(1)
    _wait_gathers(0)
    _compute(0)
    _start_scatters(0)
    _wait_scatters(0)
    _wait_gathers(1)
    _compute(1)
    _start_scatters(1)
    _wait_scatters(1)

    plsc.subcore_barrier()

    def _copy_out(nrows):
        pltpu.make_async_copy(h_sh.at[pl.ds(row0, nrows), :],
                              h_out.at[cid, pl.ds(row0, nrows), :],
                              sem_h.at[0]).start()
        pltpu.make_async_copy(s_sh.at[pl.ds(row0, nrows), :],
                              s_out.at[cid, pl.ds(row0, nrows), :],
                              sem_s.at[0]).start()
        pltpu.make_async_copy(h_sh.at[pl.ds(row0, nrows), :],
                              h_out.at[cid, pl.ds(row0, nrows), :],
                              sem_h.at[0]).wait()
        pltpu.make_async_copy(s_sh.at[pl.ds(row0, nrows), :],
                              s_out.at[cid, pl.ds(row0, nrows), :],
                              sem_s.at[0]).wait()

    @pl.when(sid < NS - 1)
    def _():
        _copy_out(RSTEP)

    @pl.when(sid == NS - 1)
    def _():
        _copy_out(400)


def _sc_edge_pass(zp, es, ed, m, edge_index):
    mesh = plsc.VectorSubcoreMesh(core_axis_name="c", subcore_axis_name="s")
    f = pl.kernel(
        _sc_body,
        out_type=(
            jax.ShapeDtypeStruct((NC, N, D), _f32),
            jax.ShapeDtypeStruct((NC, N, 16), _f32),
        ),
        mesh=mesh,
        scratch_types=[
            pltpu.VMEM_SHARED((N, D), _f32),
            pltpu.VMEM_SHARED((N, 16), _f32),
            pltpu.VMEM((NB, 2, CH), jnp.int32),
            pltpu.VMEM((NB, CH, 16), _f32),
            pltpu.VMEM((NB, CH, 16), _f32),
            pltpu.VMEM((NB, CH, D), _f32),
            pltpu.VMEM((8, 16), _f32),
            pltpu.SemaphoreType.DMA((NB,)),
            pltpu.SemaphoreType.DMA((NB,)),
            pltpu.SemaphoreType.DMA((NB,)),
            pltpu.SemaphoreType.DMA((NB,)),
            pltpu.SemaphoreType.DMA((NB,)),
            pltpu.SemaphoreType.DMA((NB,)),
        ],
        compiler_params=pltpu.CompilerParams(use_tc_tiling_on_sc=False),
    )
    return f(zp, es, ed, m, edge_index)


def _tc_post_body(hp_ref, sp_ref, t2_ref, p_ref, out_ref):
    ht = hp_ref[0] + hp_ref[1]
    st = sp_ref[0] + sp_ref[1]
    r2 = jnp.dot(1.0 / (st + 1e-16), t2_ref[...],
                 preferred_element_type=_f32,
                 precision=jax.lax.Precision.HIGHEST)
    scaled = (ht * r2).reshape(N, D)
    out_ref[...] = jnp.dot(scaled, p_ref[...],
                           preferred_element_type=_f32,
                           precision=jax.lax.Precision.HIGHEST)


def _tc_post(hp, sp, t2, perm):
    return pl.pallas_call(
        _tc_post_body,
        out_shape=jax.ShapeDtypeStruct((N, D), _f32),
    )(hp, sp, t2, perm)


@jax.jit
def kernel(x, edge_index, W, a):
    j = jnp.arange(D)
    heads = j % NUM_HEADS
    dims = j // NUM_HEADS
    perm = jax.nn.one_hot(heads * OUT_DIM + dims, D, dtype=_f32)
    mcol = jnp.arange(8 * D)
    t2 = jax.nn.one_hot(16 * (mcol // D) + (mcol % D) % NUM_HEADS,
                        D, dtype=_f32).T

    zp, es, ed, m = _tc_pre(x, W, a)
    hp, sp = _sc_edge_pass(zp, es, ed, m, edge_index)
    hp8 = hp.reshape(NC, N // 8, 8 * D)
    sp8 = sp.reshape(NC, N // 8, 8 * 16)
    return _tc_post(hp8, sp8, t2, perm)
